# baseline (speedup 1.0000x reference)
import functools

import jax
import jax.numpy as jnp
from jax import lax
from jax.experimental import pallas as pl
from jax.experimental.pallas import tpu as pltpu
from jax.experimental.pallas import tpu_sc as plsc

_D = 768
_MAX_REL = 32
_S = 512
_AROWS = 144

_SHARED_MERGED = {
    0: [(5, 4, "l"), (9, 4, "l")],
    1: [(0, 3, "h"), (9, 4, "l")],
    2: [(0, 4, "h"), (13, 3, "l")],
    3: [(0, 4, "h"), (4, 4, "h")],
}


def _clip_idx(p):
    return jnp.clip(511 - p, -_MAX_REL, _MAX_REL) + _MAX_REL


def _rpe_sc_kernel(table_hbm, out_hbm, idx_v, arena_v, c64_sh, c0_sh,
                   gsem, osem):
    nc = 2
    cid = lax.axis_index("c")
    sid = lax.axis_index("s")
    lane = lax.iota(jnp.int32, 16)
    wid = sid * nc + cid
    r = wid & 7
    q = lax.shift_right_logical(wid, 3)

    for u in range(_AROWS // 16):
        idx_v[pl.ds(0, 16)] = _clip_idx(440 + r + 16 * u + lane)
        pltpu.async_copy(
            table_hbm.at[idx_v], arena_v.at[pl.ds(16 * u, 16)], gsem
        ).wait()

    @pl.when(sid == 0)
    def _():
        for v in range(4):
            pltpu.sync_copy(
                arena_v.at[pl.ds(0, 32)], c64_sh.at[pl.ds(32 * v, 32)]
            )

    @pl.when(sid == 1)
    def _():
        for v in range(4):
            pltpu.sync_copy(
                arena_v.at[pl.ds(104, 32)], c0_sh.at[pl.ds(32 * v, 32)]
            )

    plsc.subcore_barrier()

    def fire(i, t, cq):
        s = 511 - i
        a = lax.shift_right_arithmetic(s, 3)
        shared = _SHARED_MERGED[cq]
        shared_es = {e for e0, n, _ in shared for e in range(e0, e0 + n)}
        for e0, n, route in shared:
            blk = c64_sh if route == "h" else c0_sh
            if n == 4:
                src = blk
            else:
                v = 32 * ((wid + t) & (4 - n))
                src = blk.at[pl.ds(v, 32 * n)]
            pltpu.async_copy(
                src, out_hbm.at[i, pl.ds(32 * e0, 32 * n)], osem
            )
        for e in range(16):
            if e in shared_es:
                continue
            w0 = s + 32 * e
            g = jnp.where(
                w0 >= 544, 13, jnp.where(w0 <= 448, 0, a + 4 * e - 55)
            )
            pltpu.async_copy(
                arena_v.at[pl.ds(8 * g, 32)],
                out_hbm.at[i, pl.ds(32 * e, 32)],
                osem,
            )

    def drain_slab(i):
        pltpu.make_async_copy(out_hbm.at[i], out_hbm.at[i], osem).wait()

    for cq in range(4):
        @pl.when(q == cq)
        def _(cq=cq):
            def slab_i(t):
                return 128 * cq + 8 * t + 7 - r

            def body(t, carry):
                fire(slab_i(t), t, cq)

                @pl.when(t >= 2)
                def _():
                    drain_slab(slab_i(t - 2))

                return carry

            lax.fori_loop(0, 16, body, 0)
            drain_slab(slab_i(14))
            drain_slab(slab_i(15))


def kernel(table, seq_len):
    del seq_len
    mesh = plsc.VectorSubcoreMesh(core_axis_name="c", subcore_axis_name="s")
    k = functools.partial(
        pl.kernel,
        mesh=mesh,
        out_type=jax.ShapeDtypeStruct((_S, _S, _D), jnp.float32),
        scratch_types=[
            pltpu.VMEM((16,), jnp.int32),
            pltpu.VMEM((_AROWS, _D), jnp.float32),
            pltpu.VMEM_SHARED((128, _D), jnp.float32),
            pltpu.VMEM_SHARED((128, _D), jnp.float32),
            pltpu.SemaphoreType.DMA,
            pltpu.SemaphoreType.DMA,
        ],
    )(_rpe_sc_kernel)
    return k(table)


# speedup vs baseline: 1.0361x; 1.0361x over previous
import functools

import jax
import jax.numpy as jnp
from jax import lax
from jax.experimental import pallas as pl
from jax.experimental.pallas import tpu as pltpu
from jax.experimental.pallas import tpu_sc as plsc

_D = 768
_MAX_REL = 32
_S = 512
_AROWS = 144

_SHARED_ROUTE = {
    0: {5: "l", 6: "l", 7: "l", 9: "l", 11: "l", 13: "l", 14: "l", 15: "l"},
    1: {0: "h", 1: "h", 2: "h", 9: "l", 10: "l", 11: "l", 13: "l", 15: "l"},
    2: {0: "h", 1: "h", 2: "h", 4: "h", 6: "h", 13: "l", 14: "l", 15: "l"},
    3: {0: "h", 1: "h", 2: "h", 3: "h", 4: "h", 6: "h", 8: "h", 10: "h"},
}


def _clip_idx(p):
    return jnp.clip(511 - p, -_MAX_REL, _MAX_REL) + _MAX_REL


def _rpe_sc_kernel(table_hbm, out_hbm, idx_v, arena_v, c64_sh, c0_sh,
                   gsem, osem):
    nc = 2
    cid = lax.axis_index("c")
    sid = lax.axis_index("s")
    lane = lax.iota(jnp.int32, 16)
    wid = sid * nc + cid
    r = wid & 7
    q = lax.shift_right_logical(wid, 3)

    for u in range(_AROWS // 16):
        idx_v[pl.ds(0, 16)] = _clip_idx(440 + r + 16 * u + lane)
        pltpu.async_copy(
            table_hbm.at[idx_v], arena_v.at[pl.ds(16 * u, 16)], gsem
        ).wait()

    @pl.when(sid == 0)
    def _():
        for v in range(4):
            pltpu.sync_copy(
                arena_v.at[pl.ds(0, 32)], c64_sh.at[pl.ds(32 * v, 32)]
            )

    @pl.when(sid == 1)
    def _():
        for v in range(4):
            pltpu.sync_copy(
                arena_v.at[pl.ds(104, 32)], c0_sh.at[pl.ds(32 * v, 32)]
            )

    plsc.subcore_barrier()

    def fire(i, t, cq):
        s = 511 - i
        a = lax.shift_right_arithmetic(s, 3)
        for e in range(16):
            dst = out_hbm.at[i, pl.ds(32 * e, 32)]
            route = _SHARED_ROUTE[cq].get(e)
            if route is None:
                w0 = s + 32 * e
                g = jnp.where(
                    w0 >= 544, 13, jnp.where(w0 <= 448, 0, a + 4 * e - 55)
                )
                src = arena_v.at[pl.ds(8 * g, 32)]
            else:
                v = (wid + t + e) & 3
                blk = c64_sh if route == "h" else c0_sh
                src = blk.at[pl.ds(32 * v, 32)]
            pltpu.async_copy(src, dst, osem)

    def drain_slab(i):
        pltpu.make_async_copy(out_hbm.at[i], out_hbm.at[i], osem).wait()

    for cq in range(4):
        @pl.when(q == cq)
        def _(cq=cq):
            def slab_i(t):
                return 128 * cq + 8 * t + 7 - r

            def body(t, carry):
                fire(slab_i(t), t, cq)

                @pl.when(t >= 2)
                def _():
                    drain_slab(slab_i(t - 2))

                return carry

            lax.fori_loop(0, 16, body, 0)
            drain_slab(slab_i(14))
            drain_slab(slab_i(15))


def kernel(table, seq_len):
    del seq_len
    mesh = plsc.VectorSubcoreMesh(core_axis_name="c", subcore_axis_name="s")
    k = functools.partial(
        pl.kernel,
        mesh=mesh,
        out_type=jax.ShapeDtypeStruct((_S, _S, _D), jnp.float32),
        scratch_types=[
            pltpu.VMEM((16,), jnp.int32),
            pltpu.VMEM((_AROWS, _D), jnp.float32),
            pltpu.VMEM_SHARED((128, _D), jnp.float32),
            pltpu.VMEM_SHARED((128, _D), jnp.float32),
            pltpu.SemaphoreType.DMA,
            pltpu.SemaphoreType.DMA,
        ],
    )(_rpe_sc_kernel)
    return k(table)
